# Initial kernel scaffold; baseline (speedup 1.0000x reference)
import jax
import jax.numpy as jnp
from jax.experimental import pallas as pl


def kernel(msa_idx, embed):
    raise NotImplementedError("write your pallas kernel here")


# speedup vs baseline: 1.3472x; 1.3472x over previous
import functools

import jax
import jax.numpy as jnp
from jax import lax
from jax.experimental import pallas as pl
from jax.experimental.pallas import tpu as pltpu
from jax.experimental.pallas import tpu_sc as plsc

D_MSA = 128
NUM_CORES = 2
NUM_SUBCORES = 16
NW = NUM_CORES * NUM_SUBCORES
CHUNK = 256
IDX_ROWS = CHUNK // 128


@functools.partial(jax.jit, static_argnames=("total",))
def _sc_gather(idx1d, table, *, total):
    per_w = total // NW
    steps = per_w // CHUNK
    mesh = plsc.VectorSubcoreMesh(core_axis_name="c", subcore_axis_name="s")

    @functools.partial(
        pl.kernel,
        mesh=mesh,
        out_type=jax.ShapeDtypeStruct((total, D_MSA), jnp.float32),
        scratch_types=[
            pltpu.VMEM((CHUNK,), jnp.int32),
            pltpu.VMEM((CHUNK, D_MSA), jnp.float32),
            pltpu.SemaphoreType.DMA,
        ],
    )
    def k(idx_hbm, table_hbm, out_hbm, idx_v, rows_v, sem):
        wid = lax.axis_index("s") * NUM_CORES + lax.axis_index("c")
        t_base = wid * per_w

        def body(step, carry):
            tok0 = t_base + step * CHUNK
            pltpu.sync_copy(idx_hbm.at[pl.ds(tok0, CHUNK)], idx_v)
            copies = []
            for j in range(IDX_ROWS):
                copies.append(
                    pltpu.async_copy(
                        table_hbm.at[idx_v.at[pl.ds(j * 128, 128)]],
                        rows_v.at[pl.ds(j * 128, 128)],
                        sem,
                    )
                )
            for c in copies:
                c.wait()
            pltpu.sync_copy(rows_v, out_hbm.at[pl.ds(tok0, CHUNK)])
            return carry

        lax.fori_loop(0, steps, body, 0)

    return k(idx1d, table)


def kernel(msa_idx, embed):
    if msa_idx.ndim == 2:
        msa_idx = msa_idx[None]
    b, n, l = msa_idx.shape
    total = b * n * l
    idx1d = msa_idx.reshape(total)
    out = _sc_gather(idx1d, embed, total=total)
    return out.reshape(b, n, l, D_MSA)


# speedup vs baseline: 12.6234x; 9.3700x over previous
import functools

import jax
import jax.numpy as jnp
from jax import lax
from jax.experimental import pallas as pl
from jax.experimental.pallas import tpu as pltpu
from jax.experimental.pallas import tpu_sc as plsc

D_MSA = 128
TABLE_ROWS = 32
NUM_CORES = 2
NUM_SUBCORES = 16
NW = NUM_CORES * NUM_SUBCORES
CHUNK = 256
IDX_ROWS = CHUNK // 128


@functools.partial(jax.jit, static_argnames=("total",))
def _sc_gather(idx1d, table, *, total):
    per_w = total // NW
    steps = per_w // CHUNK
    mesh = plsc.VectorSubcoreMesh(core_axis_name="c", subcore_axis_name="s")

    @functools.partial(
        pl.kernel,
        mesh=mesh,
        out_type=jax.ShapeDtypeStruct((total, D_MSA), jnp.float32),
        scratch_types=[
            pltpu.VMEM((per_w,), jnp.int32),
            pltpu.VMEM_SHARED((TABLE_ROWS, D_MSA), jnp.float32),
            pltpu.VMEM((2, CHUNK, D_MSA), jnp.float32),
            pltpu.SemaphoreType.DMA,
            pltpu.SemaphoreType.DMA,
            pltpu.SemaphoreType.DMA,
        ],
    )
    def k(idx_hbm, table_hbm, out_hbm, idx_v, table_v, rows_v, gsem, ssem0, ssem1):
        ssem = (ssem0, ssem1)
        wid = lax.axis_index("s") * NUM_CORES + lax.axis_index("c")
        t_base = wid * per_w

        @pl.when(lax.axis_index("s") == 0)
        def _stage_table():
            pltpu.sync_copy(table_hbm, table_v)

        pltpu.sync_copy(idx_hbm.at[pl.ds(t_base, per_w)], idx_v)
        plsc.subcore_barrier()

        def issue_gather(step, buf):
            for j in range(IDX_ROWS):
                pltpu.async_copy(
                    table_v.at[idx_v.at[pl.ds(step * CHUNK + j * 128, 128)]],
                    rows_v.at[buf].at[pl.ds(j * 128, 128)],
                    gsem,
                )

        def wait_gather(buf):
            for j in range(IDX_ROWS):
                pltpu.make_async_copy(
                    table_v.at[idx_v.at[pl.ds(j * 128, 128)]],
                    rows_v.at[buf].at[pl.ds(j * 128, 128)],
                    gsem,
                ).wait()

        def issue_scatter(step, buf):
            pltpu.async_copy(
                rows_v.at[buf],
                out_hbm.at[pl.ds(t_base + step * CHUNK, CHUNK)],
                ssem[buf],
            )

        def wait_scatter(buf):
            pltpu.make_async_copy(
                rows_v.at[buf], out_hbm.at[pl.ds(0, CHUNK)], ssem[buf]
            ).wait()

        def pair(i, first, last):
            s0 = 2 * i
            wait_gather(0)
            issue_scatter(s0, 0)
            if not first:
                wait_scatter(1)
            issue_gather(s0 + 1, 1)
            wait_gather(1)
            issue_scatter(s0 + 1, 1)
            wait_scatter(0)
            if not last:
                issue_gather(s0 + 2, 0)
            return i

        issue_gather(0, 0)
        pair(0, True, False)
        lax.fori_loop(1, steps // 2 - 1, lambda i, c: pair(i, False, False), 0)
        pair(steps // 2 - 1, False, True)
        wait_scatter(1)

    return k(idx1d, table)


def kernel(msa_idx, embed):
    if msa_idx.ndim == 2:
        msa_idx = msa_idx[None]
    b, n, l = msa_idx.shape
    total = b * n * l
    idx1d = msa_idx.reshape(total)
    table = jnp.zeros((TABLE_ROWS, D_MSA), embed.dtype).at[: embed.shape[0]].set(embed)
    out = _sc_gather(idx1d, table, total=total)
    return out.reshape(b, n, l, D_MSA)


# speedup vs baseline: 12.7085x; 1.0067x over previous
import functools

import jax
import jax.numpy as jnp
from jax import lax
from jax.experimental import pallas as pl
from jax.experimental.pallas import tpu as pltpu
from jax.experimental.pallas import tpu_sc as plsc

D_MSA = 128
VOCAB = 22
NUM_CORES = 2
NUM_SUBCORES = 16
NW = NUM_CORES * NUM_SUBCORES
CHUNK = 256
IDX_ROWS = CHUNK // 128


@functools.partial(jax.jit, static_argnames=("total",))
def _sc_gather(idx1d, table, *, total):
    per_w = total // NW
    steps = per_w // CHUNK
    mesh = plsc.VectorSubcoreMesh(core_axis_name="c", subcore_axis_name="s")

    @functools.partial(
        pl.kernel,
        mesh=mesh,
        out_type=jax.ShapeDtypeStruct((total, D_MSA), jnp.float32),
        scratch_types=[
            pltpu.VMEM((per_w,), jnp.int32),
            pltpu.VMEM_SHARED((VOCAB, D_MSA), jnp.float32),
            pltpu.VMEM((2, CHUNK, D_MSA), jnp.float32),
            pltpu.SemaphoreType.DMA,
            pltpu.SemaphoreType.DMA,
            pltpu.SemaphoreType.DMA,
        ],
    )
    def k(idx_hbm, table_hbm, out_hbm, idx_v, table_v, rows_v, gsem, ssem0, ssem1):
        ssem = (ssem0, ssem1)
        wid = lax.axis_index("s") * NUM_CORES + lax.axis_index("c")
        t_base = wid * per_w

        @pl.when(lax.axis_index("s") == 0)
        def _stage_table():
            pltpu.sync_copy(table_hbm, table_v)

        pltpu.sync_copy(idx_hbm.at[pl.ds(t_base, per_w)], idx_v)
        plsc.subcore_barrier()

        def issue_gather(step, buf):
            for j in range(IDX_ROWS):
                pltpu.async_copy(
                    table_v.at[idx_v.at[pl.ds(step * CHUNK + j * 128, 128)]],
                    rows_v.at[buf].at[pl.ds(j * 128, 128)],
                    gsem,
                )

        def wait_gather(buf):
            for j in range(IDX_ROWS):
                pltpu.make_async_copy(
                    table_v.at[idx_v.at[pl.ds(j * 128, 128)]],
                    rows_v.at[buf].at[pl.ds(j * 128, 128)],
                    gsem,
                ).wait()

        def issue_scatter(step, buf):
            pltpu.async_copy(
                rows_v.at[buf],
                out_hbm.at[pl.ds(t_base + step * CHUNK, CHUNK)],
                ssem[buf],
            )

        def wait_scatter(buf):
            pltpu.make_async_copy(
                rows_v.at[buf], out_hbm.at[pl.ds(0, CHUNK)], ssem[buf]
            ).wait()

        def pair(i, first, last):
            s0 = 2 * i
            wait_gather(0)
            issue_scatter(s0, 0)
            if not first:
                wait_scatter(1)
            issue_gather(s0 + 1, 1)
            wait_gather(1)
            issue_scatter(s0 + 1, 1)
            wait_scatter(0)
            if not last:
                issue_gather(s0 + 2, 0)
            return i

        issue_gather(0, 0)
        pair(0, True, False)
        lax.fori_loop(1, steps // 2 - 1, lambda i, c: pair(i, False, False), 0)
        pair(steps // 2 - 1, False, True)
        wait_scatter(1)

    return k(idx1d, table)


def kernel(msa_idx, embed):
    if msa_idx.ndim == 2:
        msa_idx = msa_idx[None]
    b, n, l = msa_idx.shape
    total = b * n * l
    idx1d = msa_idx.reshape(total)
    out = _sc_gather(idx1d, embed, total=total)
    return out.reshape(b, n, l, D_MSA)
